# baseline (speedup 1.0000x reference)
import functools

import jax
import jax.numpy as jnp
from jax import lax
from jax.experimental import pallas as pl
from jax.experimental.pallas import tpu as pltpu
from jax.experimental.pallas import tpu_sc as plsc

K = 32
MAX_RADIUS = 16.0
N = 8192
D = 64

RB = 256
NBLK = N // RB

NC = 2
NS = 16
NW = NC * NS
RPW = N // NW


G = 64
GW = N // G
TPG = 6
CW = TPG * GW


def _knn_body(xb_ref, xt_ref, nbr_ref, vmask_ref, *, row0):
    i = pl.program_id(0)
    xb = xb_ref[...]
    xt = xt_ref[...]
    srow = jnp.sum(xb * xb, axis=1, keepdims=True)
    scol = jnp.sum(xt * xt, axis=0, keepdims=True)
    prod = jax.lax.dot_general(xb, xt, (((1,), (0,)), ((), ())),
                               preferred_element_type=jnp.float32)
    d2 = srow + scol - 2.0 * prod
    rowid = (row0 + i * RB
             + jax.lax.broadcasted_iota(jnp.int32, (RB, N), 0))
    colid = jax.lax.broadcasted_iota(jnp.int32, (RB, N), 1)
    d2 = jnp.where(colid == rowid, jnp.inf, d2)

    lane = jax.lax.broadcasted_iota(jnp.int32, (RB, GW), 1)
    sv = [jnp.full((RB, GW), jnp.inf, jnp.float32) for _ in range(TPG)]
    si = [jnp.full((RB, GW), N, jnp.int32) for _ in range(TPG)]
    for g in range(G):
        v = d2[:, g * GW:(g + 1) * GW]
        vi = lane + (g * GW)
        b = [v < sv[j] for j in range(TPG)]
        for j in range(TPG - 1, 0, -1):
            sv[j] = jnp.where(b[j], jnp.where(b[j - 1], sv[j - 1], v), sv[j])
            si[j] = jnp.where(b[j], jnp.where(b[j - 1], si[j - 1], vi), si[j])
        sv[0] = jnp.where(b[0], v, sv[0])
        si[0] = jnp.where(b[0], vi, si[0])
    V0 = jnp.concatenate(sv, axis=1)
    I0 = jnp.concatenate(si, axis=1)

    def body(k, carry):
        V, accn, accv = carry
        m = jnp.min(V, axis=1, keepdims=True)
        cand = jnp.where(V == m, I0, jnp.int32(N))
        idx = jnp.min(cand, axis=1, keepdims=True)
        V = jnp.where(I0 == idx, jnp.inf, V)
        kl = jax.lax.broadcasted_iota(jnp.int32, (RB, K), 1)
        accn = jnp.where(kl == k, idx, accn)
        accv = jnp.where(kl == k, m, accv)
        return V, accn, accv

    accn0 = jnp.zeros((RB, K), jnp.int32)
    accv0 = jnp.zeros((RB, K), jnp.float32)
    _, accn, accv = lax.fori_loop(0, K, body, (V0, accn0, accv0))
    nbr_ref[...] = accn
    vmask_ref[...] = (accv < MAX_RADIUS * MAX_RADIUS).astype(jnp.float32)


def _knn_topk(x, xt, row0, rows):
    body = functools.partial(_knn_body, row0=row0)
    return pl.pallas_call(
        body,
        grid=(rows // RB,),
        in_specs=[
            pl.BlockSpec((RB, D), lambda i: (i, 0)),
            pl.BlockSpec((D, N), lambda i: (0, 0)),
        ],
        out_specs=[
            pl.BlockSpec((RB, K), lambda i: (i, 0)),
            pl.BlockSpec((RB, K), lambda i: (i, 0)),
        ],
        out_shape=[
            jax.ShapeDtypeStruct((rows, K), jnp.int32),
            jax.ShapeDtypeStruct((rows, K), jnp.float32),
        ],
    )(x, xt)


def _splat(vec16, lane):
    idx = jnp.broadcast_to(jnp.asarray(lane, jnp.int32), (16,))[:, None]
    dn = lax.GatherDimensionNumbers(offset_dims=(), collapsed_slice_dims=(0,),
                                    start_index_map=(0,))
    return lax.gather(vec16, idx, dn, (1,),
                      mode=lax.GatherScatterMode.PROMISE_IN_BOUNDS)


def _edge_body(x_hbm, xflat_hbm, nbr_hbm, vmask_hbm, pid_hbm, pidc_hbm,
               attr_hbm, y_hbm,
               idx_v, vm_v, xc_v, pidc_v,
               rows_a, rows_b, pids_a, pids_b, attr_a, attr_b, y_v,
               semx_a, semx_b, semp_a, semp_b, semo_a, semo_b, *, rpw):
    wid = lax.axis_index("s") * NC + lax.axis_index("c")
    base = wid * rpw
    pltpu.sync_copy(nbr_hbm.at[pl.ds(base * K, rpw * K)], idx_v)
    pltpu.sync_copy(vmask_hbm.at[pl.ds(base * K, rpw * K)], vm_v)
    pltpu.sync_copy(xflat_hbm.at[pl.ds(base * D, rpw * D)], xc_v)
    pltpu.sync_copy(pidc_hbm.at[pl.ds(base, rpw)], pidc_v)

    def g_start(r, rows_v, pids_v, semx, semp):
        idx_row = idx_v.at[pl.ds(r * K, K)]
        pltpu.async_copy(x_hbm.at[idx_row], rows_v, semx)
        pltpu.async_copy(pid_hbm.at[idx_row], pids_v, semp)

    def g_wait(r, rows_v, pids_v, semx, semp):
        idx_row = idx_v.at[pl.ds(r * K, K)]
        pltpu.make_async_copy(x_hbm.at[idx_row], rows_v, semx).wait()
        pltpu.make_async_copy(pid_hbm.at[idx_row], pids_v, semp).wait()

    def attr_slice(r):
        return attr_hbm.at[pl.ds((base + r) * K * 2 * D, K * 2 * D)]

    def compute(r, rows_v, pids_v, attr_v, semo):
        xc = [xc_v[pl.ds(r * D + c * 16, 16)] for c in range(D // 16)]
        vms = [vm_v[pl.ds(r * K + h * 16, 16)] for h in range(K // 16)]
        for e in range(K):
            vm = _splat(vms[e // 16], e % 16)
            for c in range(D // 16):
                a = rows_v[e, pl.ds(c * 16, 16)]
                attr_v[pl.ds(e * 2 * D + c * 16, 16)] = (a - xc[c]) * vm
                attr_v[pl.ds(e * 2 * D + D + c * 16, 16)] = (a + xc[c]) * vm
        r16 = (r // 16) * 16
        pidc = _splat(pidc_v[pl.ds(r16, 16)], r - r16)
        for h in range(K // 16):
            e0 = h * 16
            pids = pids_v[pl.ds(e0, 16)]
            ok = (pids == pidc) & (pids > 0) & (vms[h] > 0.5)
            y_v[pl.ds(r * K + e0, 16)] = jnp.where(ok, 1, 0).astype(jnp.int32)
        pltpu.async_copy(attr_v, attr_slice(r), semo)

    HR = rpw // 2
    g_start(0, rows_a, pids_a, semx_a, semp_a)

    def body(t, carry):
        r0 = 2 * t
        r1 = r0 + 1
        g_start(r1, rows_b, pids_b, semx_b, semp_b)
        g_wait(r0, rows_a, pids_a, semx_a, semp_a)

        @pl.when(t > 0)
        def _():
            pltpu.make_async_copy(attr_a, attr_slice(r0 - 2), semo_a).wait()

        compute(r0, rows_a, pids_a, attr_a, semo_a)

        @pl.when(t < HR - 1)
        def _():
            g_start(r0 + 2, rows_a, pids_a, semx_a, semp_a)

        g_wait(r1, rows_b, pids_b, semx_b, semp_b)

        @pl.when(t > 0)
        def _():
            pltpu.make_async_copy(attr_b, attr_slice(r1 - 2), semo_b).wait()

        compute(r1, rows_b, pids_b, attr_b, semo_b)
        return carry

    lax.fori_loop(0, HR, body, 0)
    pltpu.make_async_copy(attr_a, attr_slice(rpw - 2), semo_a).wait()
    pltpu.make_async_copy(attr_b, attr_slice(rpw - 1), semo_b).wait()
    pltpu.sync_copy(y_v, y_hbm.at[pl.ds(base * K, rpw * K)])


@functools.cache
def _build_edge_kernel(rows):
    rpw = rows // NW
    return pl.kernel(
        functools.partial(_edge_body, rpw=rpw),
        out_type=[
            jax.ShapeDtypeStruct((rows * K * 2 * D,), jnp.float32),
            jax.ShapeDtypeStruct((rows * K,), jnp.int32),
        ],
        mesh=plsc.VectorSubcoreMesh(core_axis_name="c", subcore_axis_name="s"),
        scratch_types=[
            pltpu.VMEM((rpw * K,), jnp.int32),
            pltpu.VMEM((rpw * K,), jnp.float32),
            pltpu.VMEM((rpw * D,), jnp.float32),
            pltpu.VMEM((rpw,), jnp.int32),
            pltpu.VMEM((K, 2 * D), jnp.float32),
            pltpu.VMEM((K, 2 * D), jnp.float32),
            pltpu.VMEM((K,), jnp.int32),
            pltpu.VMEM((K,), jnp.int32),
            pltpu.VMEM((K * 2 * D,), jnp.float32),
            pltpu.VMEM((K * 2 * D,), jnp.float32),
            pltpu.VMEM((rpw * K,), jnp.int32),
            pltpu.SemaphoreType.DMA,
            pltpu.SemaphoreType.DMA,
            pltpu.SemaphoreType.DMA,
            pltpu.SemaphoreType.DMA,
            pltpu.SemaphoreType.DMA,
            pltpu.SemaphoreType.DMA,
        ],
    )


def kernel(x, edge_index, particle_id, pt, eta, sector, reconstructable):
    xt = x.T
    pid = particle_id.astype(jnp.int32)
    xpad = jnp.concatenate([x, jnp.zeros((N, D), x.dtype)], axis=1)
    H = N // 2
    ek = _build_edge_kernel(H)
    nbr1, vm1 = _knn_topk(x[:H], xt, 0, H)
    a1, y1 = ek(xpad, x[:H].reshape(-1), nbr1.reshape(-1),
                vm1.reshape(-1), pid, pid[:H])
    nbr2, vm2 = _knn_topk(x[H:], xt, H, H)
    a2, y2 = ek(xpad, x[H:].reshape(-1), nbr2.reshape(-1),
                vm2.reshape(-1), pid, pid[H:])
    src = jnp.concatenate([nbr1.reshape(-1), nbr2.reshape(-1)])
    y = jnp.concatenate([y1, y2])
    attr_flat = jnp.concatenate([a1, a2])
    edge_attr = attr_flat.reshape(N * K, 2 * D)
    dst = jnp.broadcast_to(jnp.arange(N, dtype=src.dtype)[:, None],
                           (N, K)).reshape(-1)
    ei = jnp.stack([src, dst])
    return (x, ei, edge_index, y, pt, particle_id, sector,
            reconstructable, edge_attr, eta)


# speedup vs baseline: 1.0210x; 1.0210x over previous
import functools

import jax
import jax.numpy as jnp
from jax import lax
from jax.experimental import pallas as pl
from jax.experimental.pallas import tpu as pltpu
from jax.experimental.pallas import tpu_sc as plsc

K = 32
MAX_RADIUS = 16.0
N = 8192
D = 64

RB = 256
NBLK = N // RB

NC = 2
NS = 16
NW = NC * NS
RPW = N // NW


G = 64
GW = N // G
TPG = 6
CW = TPG * GW


def _knn_body(xb_ref, xt_ref, nbr_ref, vmask_ref, *, row0):
    i = pl.program_id(0)
    xb = xb_ref[...]
    xt = xt_ref[...]
    srow = jnp.sum(xb * xb, axis=1, keepdims=True)
    scol = jnp.sum(xt * xt, axis=0, keepdims=True)
    prod = jax.lax.dot_general(xb, xt, (((1,), (0,)), ((), ())),
                               preferred_element_type=jnp.float32)
    d2 = srow + scol - 2.0 * prod
    rowid = (row0 + i * RB
             + jax.lax.broadcasted_iota(jnp.int32, (RB, N), 0))
    colid = jax.lax.broadcasted_iota(jnp.int32, (RB, N), 1)
    d2 = jnp.where(colid == rowid, jnp.inf, d2)

    lane = jax.lax.broadcasted_iota(jnp.int32, (RB, GW), 1)
    sv = [jnp.full((RB, GW), jnp.inf, jnp.float32) for _ in range(TPG)]
    si = [jnp.full((RB, GW), N, jnp.int32) for _ in range(TPG)]
    for g in range(G):
        v = d2[:, g * GW:(g + 1) * GW]
        vi = lane + (g * GW)
        b = [v < sv[j] for j in range(TPG)]
        for j in range(TPG - 1, 0, -1):
            sv[j] = jnp.where(b[j], jnp.where(b[j - 1], sv[j - 1], v), sv[j])
            si[j] = jnp.where(b[j], jnp.where(b[j - 1], si[j - 1], vi), si[j])
        sv[0] = jnp.where(b[0], v, sv[0])
        si[0] = jnp.where(b[0], vi, si[0])
    Vt = jnp.concatenate(sv, axis=1).T
    It = jnp.concatenate(si, axis=1).T

    def body(k, carry):
        V, accn, accv = carry
        m = jnp.min(V, axis=0, keepdims=True)
        cand = jnp.where(V == m, It, jnp.int32(N))
        idx = jnp.min(cand, axis=0, keepdims=True)
        V = jnp.where(It == idx, jnp.inf, V)
        kl = jax.lax.broadcasted_iota(jnp.int32, (K, RB), 0)
        accn = jnp.where(kl == k, idx, accn)
        accv = jnp.where(kl == k, m, accv)
        return V, accn, accv

    accn0 = jnp.zeros((K, RB), jnp.int32)
    accv0 = jnp.zeros((K, RB), jnp.float32)
    _, accn, accv = lax.fori_loop(0, K, body, (Vt, accn0, accv0))
    nbr_ref[...] = accn.T
    vmask_ref[...] = (accv.T < MAX_RADIUS * MAX_RADIUS).astype(jnp.float32)


def _knn_topk(x, xt, row0, rows):
    body = functools.partial(_knn_body, row0=row0)
    return pl.pallas_call(
        body,
        grid=(rows // RB,),
        in_specs=[
            pl.BlockSpec((RB, D), lambda i: (i, 0)),
            pl.BlockSpec((D, N), lambda i: (0, 0)),
        ],
        out_specs=[
            pl.BlockSpec((RB, K), lambda i: (i, 0)),
            pl.BlockSpec((RB, K), lambda i: (i, 0)),
        ],
        out_shape=[
            jax.ShapeDtypeStruct((rows, K), jnp.int32),
            jax.ShapeDtypeStruct((rows, K), jnp.float32),
        ],
    )(x, xt)


def _splat(vec16, lane):
    idx = jnp.broadcast_to(jnp.asarray(lane, jnp.int32), (16,))[:, None]
    dn = lax.GatherDimensionNumbers(offset_dims=(), collapsed_slice_dims=(0,),
                                    start_index_map=(0,))
    return lax.gather(vec16, idx, dn, (1,),
                      mode=lax.GatherScatterMode.PROMISE_IN_BOUNDS)


def _edge_body(x_hbm, xflat_hbm, nbr_hbm, vmask_hbm, pid_hbm, pidc_hbm,
               attr_hbm, y_hbm,
               idx_v, vm_v, xc_v, pidc_v,
               rows_a, rows_b, pids_a, pids_b, attr_a, attr_b, y_v,
               semx_a, semx_b, semp_a, semp_b, semo_a, semo_b, *, rpw):
    wid = lax.axis_index("s") * NC + lax.axis_index("c")
    base = wid * rpw
    pltpu.sync_copy(nbr_hbm.at[pl.ds(base * K, rpw * K)], idx_v)
    pltpu.sync_copy(vmask_hbm.at[pl.ds(base * K, rpw * K)], vm_v)
    pltpu.sync_copy(xflat_hbm.at[pl.ds(base * D, rpw * D)], xc_v)
    pltpu.sync_copy(pidc_hbm.at[pl.ds(base, rpw)], pidc_v)

    def g_start(r, rows_v, pids_v, semx, semp):
        idx_row = idx_v.at[pl.ds(r * K, K)]
        pltpu.async_copy(x_hbm.at[idx_row], rows_v, semx)
        pltpu.async_copy(pid_hbm.at[idx_row], pids_v, semp)

    def g_wait(r, rows_v, pids_v, semx, semp):
        idx_row = idx_v.at[pl.ds(r * K, K)]
        pltpu.make_async_copy(x_hbm.at[idx_row], rows_v, semx).wait()
        pltpu.make_async_copy(pid_hbm.at[idx_row], pids_v, semp).wait()

    def attr_slice(r):
        return attr_hbm.at[pl.ds((base + r) * K * 2 * D, K * 2 * D)]

    def compute(r, rows_v, pids_v, attr_v, semo):
        xc = [xc_v[pl.ds(r * D + c * 16, 16)] for c in range(D // 16)]
        vms = [vm_v[pl.ds(r * K + h * 16, 16)] for h in range(K // 16)]
        for e in range(K):
            vm = _splat(vms[e // 16], e % 16)
            for c in range(D // 16):
                a = rows_v[e, pl.ds(c * 16, 16)]
                attr_v[pl.ds(e * 2 * D + c * 16, 16)] = (a - xc[c]) * vm
                attr_v[pl.ds(e * 2 * D + D + c * 16, 16)] = (a + xc[c]) * vm
        r16 = (r // 16) * 16
        pidc = _splat(pidc_v[pl.ds(r16, 16)], r - r16)
        for h in range(K // 16):
            e0 = h * 16
            pids = pids_v[pl.ds(e0, 16)]
            ok = (pids == pidc) & (pids > 0) & (vms[h] > 0.5)
            y_v[pl.ds(r * K + e0, 16)] = jnp.where(ok, 1, 0).astype(jnp.int32)
        pltpu.async_copy(attr_v, attr_slice(r), semo)

    HR = rpw // 2
    g_start(0, rows_a, pids_a, semx_a, semp_a)

    def body(t, carry):
        r0 = 2 * t
        r1 = r0 + 1
        g_start(r1, rows_b, pids_b, semx_b, semp_b)
        g_wait(r0, rows_a, pids_a, semx_a, semp_a)

        @pl.when(t > 0)
        def _():
            pltpu.make_async_copy(attr_a, attr_slice(r0 - 2), semo_a).wait()

        compute(r0, rows_a, pids_a, attr_a, semo_a)

        @pl.when(t < HR - 1)
        def _():
            g_start(r0 + 2, rows_a, pids_a, semx_a, semp_a)

        g_wait(r1, rows_b, pids_b, semx_b, semp_b)

        @pl.when(t > 0)
        def _():
            pltpu.make_async_copy(attr_b, attr_slice(r1 - 2), semo_b).wait()

        compute(r1, rows_b, pids_b, attr_b, semo_b)
        return carry

    lax.fori_loop(0, HR, body, 0)
    pltpu.make_async_copy(attr_a, attr_slice(rpw - 2), semo_a).wait()
    pltpu.make_async_copy(attr_b, attr_slice(rpw - 1), semo_b).wait()
    pltpu.sync_copy(y_v, y_hbm.at[pl.ds(base * K, rpw * K)])


@functools.cache
def _build_edge_kernel(rows):
    rpw = rows // NW
    return pl.kernel(
        functools.partial(_edge_body, rpw=rpw),
        out_type=[
            jax.ShapeDtypeStruct((rows * K * 2 * D,), jnp.float32),
            jax.ShapeDtypeStruct((rows * K,), jnp.int32),
        ],
        mesh=plsc.VectorSubcoreMesh(core_axis_name="c", subcore_axis_name="s"),
        scratch_types=[
            pltpu.VMEM((rpw * K,), jnp.int32),
            pltpu.VMEM((rpw * K,), jnp.float32),
            pltpu.VMEM((rpw * D,), jnp.float32),
            pltpu.VMEM((rpw,), jnp.int32),
            pltpu.VMEM((K, 2 * D), jnp.float32),
            pltpu.VMEM((K, 2 * D), jnp.float32),
            pltpu.VMEM((K,), jnp.int32),
            pltpu.VMEM((K,), jnp.int32),
            pltpu.VMEM((K * 2 * D,), jnp.float32),
            pltpu.VMEM((K * 2 * D,), jnp.float32),
            pltpu.VMEM((rpw * K,), jnp.int32),
            pltpu.SemaphoreType.DMA,
            pltpu.SemaphoreType.DMA,
            pltpu.SemaphoreType.DMA,
            pltpu.SemaphoreType.DMA,
            pltpu.SemaphoreType.DMA,
            pltpu.SemaphoreType.DMA,
        ],
    )


def kernel(x, edge_index, particle_id, pt, eta, sector, reconstructable):
    xt = x.T
    pid = particle_id.astype(jnp.int32)
    xpad = jnp.concatenate([x, jnp.zeros((N, D), x.dtype)], axis=1)
    H = N // 2
    ek = _build_edge_kernel(H)
    nbr1, vm1 = _knn_topk(x[:H], xt, 0, H)
    a1, y1 = ek(xpad, x[:H].reshape(-1), nbr1.reshape(-1),
                vm1.reshape(-1), pid, pid[:H])
    nbr2, vm2 = _knn_topk(x[H:], xt, H, H)
    a2, y2 = ek(xpad, x[H:].reshape(-1), nbr2.reshape(-1),
                vm2.reshape(-1), pid, pid[H:])
    src = jnp.concatenate([nbr1.reshape(-1), nbr2.reshape(-1)])
    y = jnp.concatenate([y1, y2])
    attr_flat = jnp.concatenate([a1, a2])
    edge_attr = attr_flat.reshape(N * K, 2 * D)
    dst = jnp.broadcast_to(jnp.arange(N, dtype=src.dtype)[:, None],
                           (N, K)).reshape(-1)
    ei = jnp.stack([src, dst])
    return (x, ei, edge_index, y, pt, particle_id, sector,
            reconstructable, edge_attr, eta)


# speedup vs baseline: 1.0309x; 1.0097x over previous
import functools

import jax
import jax.numpy as jnp
from jax import lax
from jax.experimental import pallas as pl
from jax.experimental.pallas import tpu as pltpu
from jax.experimental.pallas import tpu_sc as plsc

K = 32
MAX_RADIUS = 16.0
N = 8192
D = 64

RB = 256
NBLK = N // RB

NC = 2
NS = 16
NW = NC * NS
RPW = N // NW


G = 64
GW = N // G
TPG = 6
CW = TPG * GW


def _knn_body(xb_ref, xt_ref, nbr_ref, vmask_ref, *, row0):
    i = pl.program_id(0)
    xb = xb_ref[...]
    xt = xt_ref[...]
    srow = jnp.sum(xb * xb, axis=1, keepdims=True)
    scol = jnp.sum(xt * xt, axis=0, keepdims=True)
    prod = jax.lax.dot_general(xb, xt, (((1,), (0,)), ((), ())),
                               preferred_element_type=jnp.float32)
    d2 = srow + scol - 2.0 * prod
    rowid = (row0 + i * RB
             + jax.lax.broadcasted_iota(jnp.int32, (RB, N), 0))
    colid = jax.lax.broadcasted_iota(jnp.int32, (RB, N), 1)
    d2 = jnp.where(colid == rowid, jnp.inf, d2)

    lane = jax.lax.broadcasted_iota(jnp.int32, (RB, GW), 1)
    sv = [jnp.full((RB, GW), jnp.inf, jnp.float32) for _ in range(TPG)]
    si = [jnp.full((RB, GW), N, jnp.int32) for _ in range(TPG)]
    for g in range(G):
        v = d2[:, g * GW:(g + 1) * GW]
        vi = lane + (g * GW)
        b = [v < sv[j] for j in range(TPG)]
        for j in range(TPG - 1, 0, -1):
            sv[j] = jnp.where(b[j], jnp.where(b[j - 1], sv[j - 1], v), sv[j])
            si[j] = jnp.where(b[j], jnp.where(b[j - 1], si[j - 1], vi), si[j])
        sv[0] = jnp.where(b[0], v, sv[0])
        si[0] = jnp.where(b[0], vi, si[0])
    Vt = jnp.concatenate(sv, axis=1).T
    It = jnp.concatenate(si, axis=1).T

    def body(k, carry):
        V, accn, accv = carry
        m = jnp.min(V, axis=0, keepdims=True)
        cand = jnp.where(V == m, It, jnp.int32(N))
        idx = jnp.min(cand, axis=0, keepdims=True)
        V = jnp.where(It == idx, jnp.inf, V)
        kl = jax.lax.broadcasted_iota(jnp.int32, (K, RB), 0)
        accn = jnp.where(kl == k, idx, accn)
        accv = jnp.where(kl == k, m, accv)
        return V, accn, accv

    accn0 = jnp.zeros((K, RB), jnp.int32)
    accv0 = jnp.zeros((K, RB), jnp.float32)
    _, accn, accv = lax.fori_loop(0, K, body, (Vt, accn0, accv0))
    nbr_ref[...] = accn.T
    vmask_ref[...] = (accv.T < MAX_RADIUS * MAX_RADIUS).astype(jnp.float32)


def _knn_topk(x, xt, row0, rows):
    body = functools.partial(_knn_body, row0=row0)
    return pl.pallas_call(
        body,
        grid=(rows // RB,),
        in_specs=[
            pl.BlockSpec((RB, D), lambda i: (i, 0)),
            pl.BlockSpec((D, N), lambda i: (0, 0)),
        ],
        out_specs=[
            pl.BlockSpec((RB, K), lambda i: (i, 0)),
            pl.BlockSpec((RB, K), lambda i: (i, 0)),
        ],
        out_shape=[
            jax.ShapeDtypeStruct((rows, K), jnp.int32),
            jax.ShapeDtypeStruct((rows, K), jnp.float32),
        ],
    )(x, xt)


def _splat(vec16, lane):
    idx = jnp.broadcast_to(jnp.asarray(lane, jnp.int32), (16,))[:, None]
    dn = lax.GatherDimensionNumbers(offset_dims=(), collapsed_slice_dims=(0,),
                                    start_index_map=(0,))
    return lax.gather(vec16, idx, dn, (1,),
                      mode=lax.GatherScatterMode.PROMISE_IN_BOUNDS)


def _edge_body(x_hbm, xflat_hbm, nbr_hbm, vmask_hbm, pid_hbm, pidc_hbm,
               attr_hbm, y_hbm,
               idx_v, vm_v, xc_v, pidc_v,
               rows_a, rows_b, pids_a, pids_b, attr_a, attr_b, y_v,
               semx_a, semx_b, semp_a, semp_b, semo_a, semo_b, *, rpw):
    wid = lax.axis_index("s") * NC + lax.axis_index("c")
    base = wid * rpw
    pltpu.sync_copy(nbr_hbm.at[pl.ds(base * K, rpw * K)], idx_v)
    pltpu.sync_copy(vmask_hbm.at[pl.ds(base * K, rpw * K)], vm_v)
    pltpu.sync_copy(xflat_hbm.at[pl.ds(base * D, rpw * D)], xc_v)
    pltpu.sync_copy(pidc_hbm.at[pl.ds(base, rpw)], pidc_v)

    def g_start(r, rows_v, pids_v, semx, semp):
        idx_row = idx_v.at[pl.ds(r * K, K)]
        pltpu.async_copy(x_hbm.at[idx_row], rows_v, semx)
        pltpu.async_copy(pid_hbm.at[idx_row], pids_v, semp)

    def g_wait(r, rows_v, pids_v, semx, semp):
        idx_row = idx_v.at[pl.ds(r * K, K)]
        pltpu.make_async_copy(x_hbm.at[idx_row], rows_v, semx).wait()
        pltpu.make_async_copy(pid_hbm.at[idx_row], pids_v, semp).wait()

    def attr_slice(r):
        return attr_hbm.at[pl.ds((base + r) * K * 2 * D, K * 2 * D)]

    def compute(r, rows_v, pids_v, attr_v, semo):
        xc = [xc_v[pl.ds(r * D + c * 16, 16)] for c in range(D // 16)]
        vms = [vm_v[pl.ds(r * K + h * 16, 16)] for h in range(K // 16)]
        for e in range(K):
            vm = _splat(vms[e // 16], e % 16)
            for c in range(D // 16):
                a = rows_v[e, pl.ds(c * 16, 16)]
                attr_v[pl.ds(e * 2 * D + c * 16, 16)] = (a - xc[c]) * vm
                attr_v[pl.ds(e * 2 * D + D + c * 16, 16)] = (a + xc[c]) * vm
        r16 = (r // 16) * 16
        pidc = _splat(pidc_v[pl.ds(r16, 16)], r - r16)
        for h in range(K // 16):
            e0 = h * 16
            pids = pids_v[pl.ds(e0, 16)]
            ok = (pids == pidc) & (pids > 0) & (vms[h] > 0.5)
            y_v[pl.ds(r * K + e0, 16)] = jnp.where(ok, 1, 0).astype(jnp.int32)
        pltpu.async_copy(attr_v, attr_slice(r), semo)

    HR = rpw // 2
    g_start(0, rows_a, pids_a, semx_a, semp_a)

    def body(t, carry):
        r0 = 2 * t
        r1 = r0 + 1
        g_start(r1, rows_b, pids_b, semx_b, semp_b)
        g_wait(r0, rows_a, pids_a, semx_a, semp_a)

        @pl.when(t > 0)
        def _():
            pltpu.make_async_copy(attr_a, attr_slice(r0 - 2), semo_a).wait()

        compute(r0, rows_a, pids_a, attr_a, semo_a)

        @pl.when(t < HR - 1)
        def _():
            g_start(r0 + 2, rows_a, pids_a, semx_a, semp_a)

        g_wait(r1, rows_b, pids_b, semx_b, semp_b)

        @pl.when(t > 0)
        def _():
            pltpu.make_async_copy(attr_b, attr_slice(r1 - 2), semo_b).wait()

        compute(r1, rows_b, pids_b, attr_b, semo_b)
        return carry

    lax.fori_loop(0, HR, body, 0)
    pltpu.make_async_copy(attr_a, attr_slice(rpw - 2), semo_a).wait()
    pltpu.make_async_copy(attr_b, attr_slice(rpw - 1), semo_b).wait()
    pltpu.sync_copy(y_v, y_hbm.at[pl.ds(base * K, rpw * K)])


@functools.cache
def _build_edge_kernel(rows):
    rpw = rows // NW
    return pl.kernel(
        functools.partial(_edge_body, rpw=rpw),
        out_type=[
            jax.ShapeDtypeStruct((rows * K * 2 * D,), jnp.float32),
            jax.ShapeDtypeStruct((rows * K,), jnp.int32),
        ],
        mesh=plsc.VectorSubcoreMesh(core_axis_name="c", subcore_axis_name="s"),
        scratch_types=[
            pltpu.VMEM((rpw * K,), jnp.int32),
            pltpu.VMEM((rpw * K,), jnp.float32),
            pltpu.VMEM((rpw * D,), jnp.float32),
            pltpu.VMEM((rpw,), jnp.int32),
            pltpu.VMEM((K, 2 * D), jnp.float32),
            pltpu.VMEM((K, 2 * D), jnp.float32),
            pltpu.VMEM((K,), jnp.int32),
            pltpu.VMEM((K,), jnp.int32),
            pltpu.VMEM((K * 2 * D,), jnp.float32),
            pltpu.VMEM((K * 2 * D,), jnp.float32),
            pltpu.VMEM((rpw * K,), jnp.int32),
            pltpu.SemaphoreType.DMA,
            pltpu.SemaphoreType.DMA,
            pltpu.SemaphoreType.DMA,
            pltpu.SemaphoreType.DMA,
            pltpu.SemaphoreType.DMA,
            pltpu.SemaphoreType.DMA,
        ],
    )


def kernel(x, edge_index, particle_id, pt, eta, sector, reconstructable):
    xt = x.T
    pid = particle_id.astype(jnp.int32)
    xpad = jnp.concatenate([x, jnp.zeros((N, D), x.dtype)], axis=1)
    nbr, vm = _knn_topk(x, xt, 0, N)
    src = nbr.reshape(-1)
    attr_flat, y = _build_edge_kernel(N)(xpad, x.reshape(-1), src,
                                         vm.reshape(-1), pid, pid)
    edge_attr = attr_flat.reshape(N * K, 2 * D)
    dst = jnp.broadcast_to(jnp.arange(N, dtype=src.dtype)[:, None],
                           (N, K)).reshape(-1)
    ei = jnp.stack([src, dst])
    return (x, ei, edge_index, y, pt, particle_id, sector,
            reconstructable, edge_attr, eta)


# speedup vs baseline: 1.0557x; 1.0240x over previous
import functools

import jax
import jax.numpy as jnp
from jax import lax
from jax.experimental import pallas as pl
from jax.experimental.pallas import tpu as pltpu
from jax.experimental.pallas import tpu_sc as plsc

K = 32
MAX_RADIUS = 16.0
N = 8192
D = 64

RB = 256
NBLK = N // RB

NC = 2
NS = 16
NW = NC * NS
RPW = N // NW


G = 64
GW = N // G
TPG = 6
CW = TPG * GW


def _knn_body(xb_ref, xt_ref, nbr_ref, vmask_ref, scol_ref, *, row0):
    i = pl.program_id(0)
    xb = xb_ref[...]
    xt = xt_ref[...]
    srow = jnp.sum(xb * xb, axis=1, keepdims=True)

    @pl.when(i == 0)
    def _():
        scol_ref[...] = jnp.sum(xt * xt, axis=0, keepdims=True)

    scol = scol_ref[...]
    prod = jax.lax.dot_general(xb, xt, (((1,), (0,)), ((), ())),
                               preferred_element_type=jnp.float32)
    d2 = srow + scol - 2.0 * prod
    rowid = (row0 + i * RB
             + jax.lax.broadcasted_iota(jnp.int32, (RB, N), 0))
    colid = jax.lax.broadcasted_iota(jnp.int32, (RB, N), 1)
    d2 = jnp.where(colid == rowid, jnp.inf, d2)

    lane = jax.lax.broadcasted_iota(jnp.int32, (RB, GW), 1)
    sv = [jnp.full((RB, GW), jnp.inf, jnp.float32) for _ in range(TPG)]
    si = [jnp.full((RB, GW), N, jnp.int32) for _ in range(TPG)]
    for g in range(G):
        v = d2[:, g * GW:(g + 1) * GW]
        vi = lane + (g * GW)
        b = [v < sv[j] for j in range(TPG)]
        for j in range(TPG - 1, 0, -1):
            sv[j] = jnp.where(b[j], jnp.where(b[j - 1], sv[j - 1], v), sv[j])
            si[j] = jnp.where(b[j], jnp.where(b[j - 1], si[j - 1], vi), si[j])
        sv[0] = jnp.where(b[0], v, sv[0])
        si[0] = jnp.where(b[0], vi, si[0])
    Vt = jnp.concatenate(sv, axis=1).T
    It = jnp.concatenate(si, axis=1).T

    def body(k, carry):
        V, accn, accv = carry
        m = jnp.min(V, axis=0, keepdims=True)
        cand = jnp.where(V == m, It, jnp.int32(N))
        idx = jnp.min(cand, axis=0, keepdims=True)
        V = jnp.where(It == idx, jnp.inf, V)
        kl = jax.lax.broadcasted_iota(jnp.int32, (K, RB), 0)
        accn = jnp.where(kl == k, idx, accn)
        accv = jnp.where(kl == k, m, accv)
        return V, accn, accv

    accn0 = jnp.zeros((K, RB), jnp.int32)
    accv0 = jnp.zeros((K, RB), jnp.float32)
    _, accn, accv = lax.fori_loop(0, K, body, (Vt, accn0, accv0))
    nbr_ref[...] = accn.T
    vmask_ref[...] = (accv.T < MAX_RADIUS * MAX_RADIUS).astype(jnp.float32)


def _knn_topk(x, xt, row0, rows):
    body = functools.partial(_knn_body, row0=row0)
    return pl.pallas_call(
        body,
        grid=(rows // RB,),
        in_specs=[
            pl.BlockSpec((RB, D), lambda i: (i, 0)),
            pl.BlockSpec((D, N), lambda i: (0, 0)),
        ],
        out_specs=[
            pl.BlockSpec((RB, K), lambda i: (i, 0)),
            pl.BlockSpec((RB, K), lambda i: (i, 0)),
        ],
        out_shape=[
            jax.ShapeDtypeStruct((rows, K), jnp.int32),
            jax.ShapeDtypeStruct((rows, K), jnp.float32),
        ],
        scratch_shapes=[pltpu.VMEM((1, N), jnp.float32)],
    )(x, xt)


def _splat(vec16, lane):
    idx = jnp.broadcast_to(jnp.asarray(lane, jnp.int32), (16,))[:, None]
    dn = lax.GatherDimensionNumbers(offset_dims=(), collapsed_slice_dims=(0,),
                                    start_index_map=(0,))
    return lax.gather(vec16, idx, dn, (1,),
                      mode=lax.GatherScatterMode.PROMISE_IN_BOUNDS)


def _edge_body(x_hbm, xflat_hbm, nbr_hbm, vmask_hbm, pid_hbm, pidc_hbm,
               attr_hbm, y_hbm,
               idx_v, vm_v, xc_v, pidc_v,
               rows_a, rows_b, pids_a, pids_b, attr_a, attr_b, y_v,
               semx_a, semx_b, semp_a, semp_b, semo_a, semo_b, *, rpw):
    wid = lax.axis_index("s") * NC + lax.axis_index("c")
    base = wid * rpw
    pltpu.sync_copy(nbr_hbm.at[pl.ds(base * K, rpw * K)], idx_v)
    pltpu.sync_copy(vmask_hbm.at[pl.ds(base * K, rpw * K)], vm_v)
    pltpu.sync_copy(xflat_hbm.at[pl.ds(base * D, rpw * D)], xc_v)
    pltpu.sync_copy(pidc_hbm.at[pl.ds(base, rpw)], pidc_v)

    RPB = 2
    BK = RPB * K

    def g_start(p, rows_v, pids_v, semx, semp):
        idx_row = idx_v.at[pl.ds(p * BK, BK)]
        pltpu.async_copy(x_hbm.at[idx_row], rows_v, semx)
        pltpu.async_copy(pid_hbm.at[idx_row], pids_v, semp)

    def g_wait(p, rows_v, pids_v, semx, semp):
        idx_row = idx_v.at[pl.ds(p * BK, BK)]
        pltpu.make_async_copy(x_hbm.at[idx_row], rows_v, semx).wait()
        pltpu.make_async_copy(pid_hbm.at[idx_row], pids_v, semp).wait()

    def attr_slice(p):
        return attr_hbm.at[pl.ds((base * K + p * BK) * 2 * D, BK * 2 * D)]

    def compute(p, rows_v, pids_v, attr_v, semo):
        for s in range(RPB):
            r = p * RPB + s
            xc = [xc_v[pl.ds(r * D + c * 16, 16)] for c in range(D // 16)]
            vms = [vm_v[pl.ds(r * K + h * 16, 16)] for h in range(K // 16)]
            for e in range(K):
                vm = _splat(vms[e // 16], e % 16)
                eo = (s * K + e) * 2 * D
                for c in range(D // 16):
                    a = rows_v[s * K + e, pl.ds(c * 16, 16)]
                    attr_v[pl.ds(eo + c * 16, 16)] = (a - xc[c]) * vm
                    attr_v[pl.ds(eo + D + c * 16, 16)] = (a + xc[c]) * vm
            r16 = (r // 16) * 16
            pidc = _splat(pidc_v[pl.ds(r16, 16)], r - r16)
            for h in range(K // 16):
                e0 = h * 16
                pids = pids_v[pl.ds(s * K + e0, 16)]
                ok = (pids == pidc) & (pids > 0) & (vms[h] > 0.5)
                y_v[pl.ds(r * K + e0, 16)] = (
                    jnp.where(ok, 1, 0).astype(jnp.int32))
        pltpu.async_copy(attr_v, attr_slice(p), semo)

    NB = rpw // RPB
    HB = NB // 2
    g_start(0, rows_a, pids_a, semx_a, semp_a)

    def body(t, carry):
        p0 = 2 * t
        p1 = p0 + 1
        g_start(p1, rows_b, pids_b, semx_b, semp_b)
        g_wait(p0, rows_a, pids_a, semx_a, semp_a)

        @pl.when(t > 0)
        def _():
            pltpu.make_async_copy(attr_a, attr_slice(p0 - 2), semo_a).wait()

        compute(p0, rows_a, pids_a, attr_a, semo_a)

        @pl.when(t < HB - 1)
        def _():
            g_start(p0 + 2, rows_a, pids_a, semx_a, semp_a)

        g_wait(p1, rows_b, pids_b, semx_b, semp_b)

        @pl.when(t > 0)
        def _():
            pltpu.make_async_copy(attr_b, attr_slice(p1 - 2), semo_b).wait()

        compute(p1, rows_b, pids_b, attr_b, semo_b)
        return carry

    lax.fori_loop(0, HB, body, 0)
    pltpu.make_async_copy(attr_a, attr_slice(NB - 2), semo_a).wait()
    pltpu.make_async_copy(attr_b, attr_slice(NB - 1), semo_b).wait()
    pltpu.sync_copy(y_v, y_hbm.at[pl.ds(base * K, rpw * K)])


@functools.cache
def _build_edge_kernel(rows):
    rpw = rows // NW
    return pl.kernel(
        functools.partial(_edge_body, rpw=rpw),
        out_type=[
            jax.ShapeDtypeStruct((rows * K * 2 * D,), jnp.float32),
            jax.ShapeDtypeStruct((rows * K,), jnp.int32),
        ],
        mesh=plsc.VectorSubcoreMesh(core_axis_name="c", subcore_axis_name="s"),
        scratch_types=[
            pltpu.VMEM((rpw * K,), jnp.int32),
            pltpu.VMEM((rpw * K,), jnp.float32),
            pltpu.VMEM((rpw * D,), jnp.float32),
            pltpu.VMEM((rpw,), jnp.int32),
            pltpu.VMEM((2 * K, 2 * D), jnp.float32),
            pltpu.VMEM((2 * K, 2 * D), jnp.float32),
            pltpu.VMEM((2 * K,), jnp.int32),
            pltpu.VMEM((2 * K,), jnp.int32),
            pltpu.VMEM((2 * K * 2 * D,), jnp.float32),
            pltpu.VMEM((2 * K * 2 * D,), jnp.float32),
            pltpu.VMEM((rpw * K,), jnp.int32),
            pltpu.SemaphoreType.DMA,
            pltpu.SemaphoreType.DMA,
            pltpu.SemaphoreType.DMA,
            pltpu.SemaphoreType.DMA,
            pltpu.SemaphoreType.DMA,
            pltpu.SemaphoreType.DMA,
        ],
    )


def kernel(x, edge_index, particle_id, pt, eta, sector, reconstructable):
    xt = x.T
    pid = particle_id.astype(jnp.int32)
    xpad = jnp.concatenate([x, jnp.zeros((N, D), x.dtype)], axis=1)
    nbr, vm = _knn_topk(x, xt, 0, N)
    src = nbr.reshape(-1)
    attr_flat, y = _build_edge_kernel(N)(xpad, x.reshape(-1), src,
                                         vm.reshape(-1), pid, pid)
    edge_attr = attr_flat.reshape(N * K, 2 * D)
    dst = jnp.broadcast_to(jnp.arange(N, dtype=src.dtype)[:, None],
                           (N, K)).reshape(-1)
    ei = jnp.stack([src, dst])
    return (x, ei, edge_index, y, pt, particle_id, sector,
            reconstructable, edge_attr, eta)
